# Initial kernel scaffold; baseline (speedup 1.0000x reference)
import jax
import jax.numpy as jnp
from jax.experimental import pallas as pl


def kernel(temporal_order, grid_shape, temporal_table, qubit_table):
    raise NotImplementedError("write your pallas kernel here")


# speedup vs baseline: 2.8328x; 2.8328x over previous
import functools

import jax
import jax.numpy as jnp
from jax import lax
from jax.experimental import pallas as pl
from jax.experimental.pallas import tpu as pltpu
from jax.experimental.pallas import tpu_sc as plsc

D_MODEL = 128
HALF = D_MODEL // 2
N = 327680

NC = 2
NS = 16
NW = NC * NS
ROWS_PER_W = N // NW
CHUNK = 512
SUB = CHUNK // 128
N_CHUNKS = ROWS_PER_W // CHUNK


def _sc_body(torder_hbm, ttable_hbm, qtable_hbm, nq_hbm, out_hbm,
             tidx_v, qidx_v, trows_v, qrows_v, nq_v, sem):
    wid = lax.axis_index("s") * NC + lax.axis_index("c")
    wbase = wid * ROWS_PER_W

    pltpu.sync_copy(nq_hbm, nq_v)
    nq_i = nq_v[...]
    nq_f = nq_i.astype(jnp.float32)
    lane = jax.lax.iota(jnp.int32, 16)

    def chunk_body(ci, carry):
        base = wbase + ci * CHUNK

        for j in range(SUB):
            pltpu.sync_copy(torder_hbm.at[pl.ds(base + j * 128, 128)],
                            tidx_v.at[j])

        for j in range(SUB):
            for k in range(8):
                ivec = base + (j * 128 + k * 16) + lane
                t = (ivec.astype(jnp.float32) / nq_f).astype(jnp.int32)
                r = ivec - t * nq_i
                r = jnp.where(r < 0, r + nq_i, r)
                r = jnp.where(r >= nq_i, r - nq_i, r)
                qidx_v[j, pl.ds(k * 16, 16)] = r

        copies = []
        for j in range(SUB):
            copies.append(pltpu.async_copy(
                ttable_hbm.at[tidx_v.at[j]],
                trows_v.at[pl.ds(j * 128, 128)], sem))
            copies.append(pltpu.async_copy(
                qtable_hbm.at[qidx_v.at[j]],
                qrows_v.at[pl.ds(j * 128, 128)], sem))
        for c in copies:
            c.wait()

        pltpu.sync_copy(trows_v,
                        out_hbm.at[pl.ds(base, CHUNK), pl.ds(0, HALF)])
        pltpu.sync_copy(qrows_v,
                        out_hbm.at[pl.ds(base, CHUNK), pl.ds(HALF, HALF)])
        return carry

    lax.fori_loop(0, N_CHUNKS, chunk_body, 0)


@jax.jit
def _call(temporal_order, temporal_table, qubit_table, nq16):
    mesh = plsc.VectorSubcoreMesh(core_axis_name="c", subcore_axis_name="s")
    f = pl.kernel(
        _sc_body,
        mesh=mesh,
        compiler_params=pltpu.CompilerParams(use_tc_tiling_on_sc=False),
        out_type=jax.ShapeDtypeStruct((N, D_MODEL), jnp.float32),
        scratch_types=[
            pltpu.VMEM((SUB, 128), jnp.int32),
            pltpu.VMEM((SUB, 128), jnp.int32),
            pltpu.VMEM((CHUNK, HALF), jnp.float32),
            pltpu.VMEM((CHUNK, HALF), jnp.float32),
            pltpu.VMEM((16,), jnp.int32),
            pltpu.SemaphoreType.DMA,
        ],
    )
    return f(temporal_order, temporal_table, qubit_table, nq16)


def kernel(temporal_order, grid_shape, temporal_table, qubit_table):
    nq16 = jnp.broadcast_to(grid_shape[1].astype(jnp.int32), (16,))
    return _call(temporal_order.astype(jnp.int32), temporal_table,
                 qubit_table, nq16)
